# baseline (speedup 1.0000x reference)
import functools

import jax
import jax.numpy as jnp
from jax.experimental import pallas as pl
from jax.experimental.pallas import tpu as pltpu

E = 8
K = 2
D = 768
EP = 128
BM = 1024


def _moe_block(x_ref, wg_ref, we_ref, be_ref, o_ref):
    x_b = x_ref[...]
    logits = jnp.dot(x_b, wg_ref[...],
                     preferred_element_type=jnp.float32)
    lane = jax.lax.broadcasted_iota(jnp.int32, logits.shape, 1)
    logits = jnp.where(lane < E, logits, -1e30)

    v0 = jnp.max(logits, axis=1, keepdims=True)
    a0 = jnp.min(jnp.where(logits == v0, lane, EP), axis=1,
                 keepdims=True)
    logits2 = jnp.where(lane == a0, -1e30, logits)
    v1 = jnp.max(logits2, axis=1, keepdims=True)
    a1 = jnp.min(jnp.where(logits2 == v1, lane, EP), axis=1,
                 keepdims=True)

    w0 = 1.0 / (1.0 + jnp.exp(v1 - v0))
    w1 = 1.0 - w0

    acc = jnp.zeros((x_b.shape[0], D), dtype=jnp.float32)
    for e in range(E):
        w_e = jnp.where(a0 == e, w0, 0.0) + jnp.where(a1 == e, w1, 0.0)
        acc = acc + w_e * (jnp.dot(x_b, we_ref[e],
                                   preferred_element_type=jnp.float32)
                           + be_ref[e][None, :])
    o_ref[...] = acc


@jax.jit
def _moe(xs, wg_pad, We, be):
    T = xs.shape[0]
    grid = (T // BM,)
    return pl.pallas_call(
        _moe_block,
        grid=grid,
        in_specs=[
            pl.BlockSpec((BM, D), lambda i: (i, 0)),
            pl.BlockSpec((D, EP), lambda i: (0, 0)),
            pl.BlockSpec((E, D, D), lambda i: (0, 0, 0)),
            pl.BlockSpec((E, D), lambda i: (0, 0)),
        ],
        out_specs=pl.BlockSpec((BM, D), lambda i: (i, 0)),
        out_shape=jax.ShapeDtypeStruct((T, D), jnp.float32),
    )(xs, wg_pad, We, be)


def kernel(x, Wg, We, be):
    xs = x.reshape(-1, x.shape[-1])
    wg_pad = jnp.pad(Wg, ((0, 0), (0, EP - Wg.shape[1])))
    out = _moe(xs, wg_pad, We, be)
    return out.reshape(x.shape)


# speedup vs baseline: 1.0021x; 1.0021x over previous
import functools

import jax
import jax.numpy as jnp
from jax.experimental import pallas as pl
from jax.experimental.pallas import tpu as pltpu

E = 8
K = 2
D = 768
EP = 128
BM = 512


def _moe_block(x_ref, wg_ref, we_ref, be_ref, o_ref):
    x_b = x_ref[...]
    logits = jnp.dot(x_b, wg_ref[...],
                     preferred_element_type=jnp.float32)
    lane = jax.lax.broadcasted_iota(jnp.int32, logits.shape, 1)
    logits = jnp.where(lane < E, logits, -1e30)

    v0 = jnp.max(logits, axis=1, keepdims=True)
    a0 = jnp.min(jnp.where(logits == v0, lane, EP), axis=1,
                 keepdims=True)
    logits2 = jnp.where(lane == a0, -1e30, logits)
    v1 = jnp.max(logits2, axis=1, keepdims=True)
    a1 = jnp.min(jnp.where(logits2 == v1, lane, EP), axis=1,
                 keepdims=True)

    w0 = 1.0 / (1.0 + jnp.exp(v1 - v0))
    w1 = 1.0 - w0

    acc = jnp.zeros((x_b.shape[0], D), dtype=jnp.float32)
    for e in range(E):
        w_e = jnp.where(a0 == e, w0, 0.0) + jnp.where(a1 == e, w1, 0.0)
        acc = acc + w_e * (jnp.dot(x_b, we_ref[e],
                                   preferred_element_type=jnp.float32)
                           + be_ref[e][None, :])
    o_ref[...] = acc


@jax.jit
def _moe(xs, wg_pad, We, be):
    T = xs.shape[0]
    grid = (T // BM,)
    return pl.pallas_call(
        _moe_block,
        grid=grid,
        in_specs=[
            pl.BlockSpec((BM, D), lambda i: (i, 0)),
            pl.BlockSpec((D, EP), lambda i: (0, 0)),
            pl.BlockSpec((E, D, D), lambda i: (0, 0, 0)),
            pl.BlockSpec((E, D), lambda i: (0, 0)),
        ],
        out_specs=pl.BlockSpec((BM, D), lambda i: (i, 0)),
        out_shape=jax.ShapeDtypeStruct((T, D), jnp.float32),
    )(xs, wg_pad, We, be)


def kernel(x, Wg, We, be):
    xs = x.reshape(-1, x.shape[-1])
    wg_pad = jnp.pad(Wg, ((0, 0), (0, EP - Wg.shape[1])))
    out = _moe(xs, wg_pad, We, be)
    return out.reshape(x.shape)
